# baseline (speedup 1.0000x reference)
import functools

import jax
import jax.numpy as jnp
from jax import lax
from jax.experimental import pallas as pl
from jax.experimental.pallas import tpu as pltpu
from jax.experimental.pallas import tpu_sc as plsc

_GAMMA = 12.0
_B = 16384
_DIM = 64
_NC = 2
_NS = 16
_NW = _NC * _NS
_BPW = _B // _NW
_NCHUNK = 4
_CHUNK = _BPW // _NCHUNK
_GROUPS = _BPW // 16


def _make_kernel():
    mesh = plsc.VectorSubcoreMesh(
        core_axis_name="c", subcore_axis_name="s",
        num_cores=_NC, num_subcores=_NS,
    )

    @functools.partial(
        pl.kernel,
        out_type=jax.ShapeDtypeStruct((_NW, _BPW), jnp.float32),
        mesh=mesh,
        compiler_params=pltpu.CompilerParams(
            needs_layout_passes=False, use_tc_tiling_on_sc=False),
        scratch_types=[
            pltpu.VMEM((_BPW * 3,), jnp.int32),
            pltpu.VMEM((_NCHUNK, _CHUNK), jnp.int32),
            pltpu.VMEM((_NCHUNK, _CHUNK), jnp.int32),
            pltpu.VMEM((_NCHUNK, _CHUNK), jnp.int32),
            pltpu.VMEM((_NCHUNK, _CHUNK, _DIM), jnp.float32),
            pltpu.VMEM((_NCHUNK, _CHUNK, _DIM), jnp.float32),
            pltpu.VMEM((_NCHUNK, _CHUNK, _DIM), jnp.float32),
            pltpu.VMEM((16, 17), jnp.float32),
            pltpu.VMEM((_BPW,), jnp.float32),
            pltpu.SemaphoreType.DMA,
        ],
    )
    def kge_score(samp_hbm, ent_hbm, rel_hbm, out_hbm,
                  samp_v, hidx, ridx, tidx, hrow, rrow, trow, wtile, out_v,
                  sem):
        wid = lax.axis_index("s") * _NC + lax.axis_index("c")

        pltpu.sync_copy(samp_hbm.at[wid], samp_v)

        lanes = lax.iota(jnp.int32, 16)
        col_dst = (hidx, ridx, tidx)
        for g in range(_GROUPS):
            j, r0 = divmod(g * 16, _CHUNK)
            for c in range(3):
                v = plsc.load_gather(samp_v, [lanes * 3 + (g * 48 + c)])
                col_dst[c][j, pl.ds(r0, 16)] = v

        copies = []
        for j in range(_NCHUNK):
            copies.append(pltpu.async_copy(ent_hbm.at[hidx.at[j]], hrow.at[j], sem))
            copies.append(pltpu.async_copy(rel_hbm.at[ridx.at[j]], rrow.at[j], sem))
            copies.append(pltpu.async_copy(ent_hbm.at[tidx.at[j]], trow.at[j], sem))
        for cp in copies:
            cp.wait()

        def group_body(g, carry):
            j = g // (_CHUNK // 16)
            r0 = (g % (_CHUNK // 16)) * 16
            for row in range(16):
                acc = jnp.zeros((16,), jnp.float32)
                for c in range(_DIM // 16):
                    hv = hrow[j, r0 + row, pl.ds(c * 16, 16)]
                    rv = rrow[j, r0 + row, pl.ds(c * 16, 16)]
                    tv = trow[j, r0 + row, pl.ds(c * 16, 16)]
                    acc = acc + jnp.abs(hv + rv - tv)
                wtile[row, pl.ds(0, 16)] = acc
            tot = jnp.zeros((16,), jnp.float32)
            for d in range(16):
                tot = tot + plsc.load_gather(
                    wtile, [lanes, jnp.full((16,), d, jnp.int32)])
            out_v[pl.ds(g * 16, 16)] = _GAMMA - tot
            return carry

        lax.fori_loop(0, _GROUPS, group_body, 0)

        pltpu.sync_copy(out_v, out_hbm.at[wid])

    return kge_score


_kge_score = _make_kernel()


def kernel(sample, entity_embedding, relation_embedding):
    samp = sample.astype(jnp.int32).reshape(_NW, _BPW * 3)
    out = _kge_score(samp, entity_embedding, relation_embedding)
    return out.reshape(_B, 1)
